# baseline (speedup 1.0000x reference)
import jax
import jax.numpy as jnp
from jax import lax
from jax.experimental import pallas as pl
from jax.experimental.pallas import tpu as pltpu
from jax.experimental.pallas import tpu_sc as plsc

N_NODES = 10000
N_EDGES = 320000
D = 128

CHUNK = 128
NC = 2
NS = 16
N_TILES = NC * NS
CPT = 80
HALF = CPT // 2
N_CHUNKS = CPT * N_TILES
PAD_DST = N_NODES
N_ACC = 10112
ROWS_PER_SUB = N_ACC // NS

_mesh = plsc.VectorSubcoreMesh(core_axis_name="core", subcore_axis_name="subcore")


def _sc_body(x_hbm, src_hbm, dst_hbm, part_hbm,
             acc, src_idx, dst_idx, rows0, rows1,
             sem_s, sem_d, sem0, sem1, sem_sc0, sem_sc1):
    c = lax.axis_index("core")
    s = lax.axis_index("subcore")
    t = c * NS + s

    cp_s = pltpu.async_copy(src_hbm.at[pl.ds(t * CPT, HALF)], src_idx, sem_s)
    cp_d = pltpu.async_copy(dst_hbm.at[pl.ds(t * CPT, HALF)], dst_idx, sem_d)

    zero16 = jnp.zeros((16,), jnp.float32)

    @pl.loop(0, CHUNK)
    def _zero_rows(i):
        for j in range(0, D, 16):
            rows0[i, pl.ds(j, 16)] = zero16

    for k in range(ROWS_PER_SUB // CHUNK):
        pltpu.sync_copy(rows0, acc.at[pl.ds(s * ROWS_PER_SUB + k * CHUNK, CHUNK)])
    _tail = ROWS_PER_SUB % CHUNK
    if _tail:
        pltpu.sync_copy(
            rows0.at[pl.ds(0, _tail)],
            acc.at[pl.ds(s * ROWS_PER_SUB + ROWS_PER_SUB - _tail, _tail)])

    plsc.subcore_barrier()
    cp_s.wait()
    cp_d.wait()

    for h in range(CPT // HALF):
        if h > 0:
            pltpu.sync_copy(src_hbm.at[pl.ds(t * CPT + h * HALF, HALF)], src_idx)
            pltpu.sync_copy(dst_hbm.at[pl.ds(t * CPT + h * HALF, HALF)], dst_idx)

        pltpu.async_copy(x_hbm.at[src_idx.at[0]], rows0, sem0)
        pltpu.async_copy(x_hbm.at[src_idx.at[1]], rows1, sem1)

        @pl.loop(0, HALF, step=2)
        def _edges(i):
            pltpu.make_async_copy(x_hbm.at[src_idx.at[i]], rows0, sem0).wait()
            pltpu.async_copy(rows0, acc.at[dst_idx.at[i]], sem_sc0, add=True)

            pltpu.make_async_copy(x_hbm.at[src_idx.at[i + 1]], rows1, sem1).wait()
            pltpu.async_copy(rows1, acc.at[dst_idx.at[i + 1]], sem_sc1, add=True)

            pltpu.make_async_copy(rows0, acc.at[dst_idx.at[i]], sem_sc0).wait()

            @pl.when(i + 2 < HALF)
            def _():
                pltpu.async_copy(x_hbm.at[src_idx.at[i + 2]], rows0, sem0)

            pltpu.make_async_copy(rows1, acc.at[dst_idx.at[i + 1]], sem_sc1).wait()

            @pl.when(i + 3 < HALF)
            def _():
                pltpu.async_copy(x_hbm.at[src_idx.at[i + 3]], rows1, sem1)

    plsc.subcore_barrier()

    pltpu.sync_copy(acc.at[pl.ds(s * ROWS_PER_SUB, ROWS_PER_SUB)],
                    part_hbm.at[c].at[pl.ds(s * ROWS_PER_SUB, ROWS_PER_SUB)])


_sc_aggregate = pl.kernel(
    _sc_body,
    out_type=jax.ShapeDtypeStruct((NC, N_ACC, D), jnp.float32),
    mesh=_mesh,
    scratch_types=[
        pltpu.VMEM_SHARED((N_ACC, D), jnp.float32),
        pltpu.VMEM((HALF, CHUNK), jnp.int32),
        pltpu.VMEM((HALF, CHUNK), jnp.int32),
        pltpu.VMEM((CHUNK, D), jnp.float32),
        pltpu.VMEM((CHUNK, D), jnp.float32),
        pltpu.SemaphoreType.DMA,
        pltpu.SemaphoreType.DMA,
        pltpu.SemaphoreType.DMA,
        pltpu.SemaphoreType.DMA,
        pltpu.SemaphoreType.DMA,
        pltpu.SemaphoreType.DMA,
    ],
)


def _combine_body(p0, p1, w, o):
    o[...] = (p0[0] + p1[0]) * w[...]


def _combine(part, w2d):
    blk = 1000
    return pl.pallas_call(
        _combine_body,
        out_shape=jax.ShapeDtypeStruct((N_NODES, D), jnp.float32),
        grid=(N_NODES // blk,),
        in_specs=[
            pl.BlockSpec((1, blk, D), lambda i: (0, i, 0)),
            pl.BlockSpec((1, blk, D), lambda i: (1, i, 0)),
            pl.BlockSpec((1, D), lambda i: (0, 0)),
        ],
        out_specs=pl.BlockSpec((blk, D), lambda i: (i, 0)),
    )(part, part, w2d)


def kernel(feature_matrix, edge_index, W):
    n_pad = N_CHUNKS * CHUNK - N_EDGES
    src = jnp.concatenate(
        [edge_index[0], jnp.zeros((n_pad,), jnp.int32)]).reshape(N_CHUNKS, CHUNK)
    dst = jnp.concatenate(
        [edge_index[1], jnp.full((n_pad,), PAD_DST, jnp.int32)]).reshape(N_CHUNKS, CHUNK)
    part = _sc_aggregate(feature_matrix, src, dst)
    return _combine(part, W.reshape(1, D))


# speedup vs baseline: 3.0345x; 3.0345x over previous
import jax
import jax.numpy as jnp
from jax import lax
from jax.experimental import pallas as pl
from jax.experimental.pallas import tpu as pltpu
from jax.experimental.pallas import tpu_sc as plsc

N_NODES = 10000
N_EDGES = 320000
D = 128

CHUNK = 128
NC = 2
NS = 16
N_TILES = NC * NS
CPT = 80
HALF = CPT // 2
N_CHUNKS = CPT * N_TILES
N_REAL = N_EDGES // CHUNK
N_ACC = 10112
ROWS_PER_SUB = N_ACC // NS

_mesh = plsc.VectorSubcoreMesh(core_axis_name="core", subcore_axis_name="subcore")


def _sc_body(x_hbm, src_hbm, dst_hbm, part_hbm,
             acc, src_idx, dst_idx, rows0, rows1,
             sem_s, sem_d, sem0, sem1, sem_sc0, sem_sc1):
    c = lax.axis_index("core")
    s = lax.axis_index("subcore")
    t = c * NS + s

    cp_s = pltpu.async_copy(src_hbm.at[pl.ds(t * CPT, HALF)], src_idx, sem_s)
    cp_d = pltpu.async_copy(dst_hbm.at[pl.ds(t * CPT, HALF)], dst_idx, sem_d)

    zero16 = jnp.zeros((16,), jnp.float32)

    @pl.loop(0, CHUNK)
    def _zero_rows(i):
        for j in range(0, D, 16):
            rows0[i, pl.ds(j, 16)] = zero16

    for k in range(ROWS_PER_SUB // CHUNK):
        pltpu.sync_copy(rows0, acc.at[pl.ds(s * ROWS_PER_SUB + k * CHUNK, CHUNK)])
    _tail = ROWS_PER_SUB % CHUNK
    if _tail:
        pltpu.sync_copy(
            rows0.at[pl.ds(0, _tail)],
            acc.at[pl.ds(s * ROWS_PER_SUB + ROWS_PER_SUB - _tail, _tail)])

    plsc.subcore_barrier()
    cp_s.wait()
    cp_d.wait()

    for h in range(CPT // HALF):
        m = jnp.clip(N_REAL - (t * CPT + h * HALF), 0, HALF)

        if h > 0:
            @pl.when(m > 0)
            def _reload():
                pltpu.sync_copy(src_hbm.at[pl.ds(t * CPT + h * HALF, HALF)],
                                src_idx)
                pltpu.sync_copy(dst_hbm.at[pl.ds(t * CPT + h * HALF, HALF)],
                                dst_idx)

        @pl.when(0 < m)
        def _pr0():
            pltpu.async_copy(x_hbm.at[src_idx.at[0]], rows0, sem0)

        @pl.when(1 < m)
        def _pr1():
            pltpu.async_copy(x_hbm.at[src_idx.at[1]], rows1, sem1)

        @pl.loop(0, HALF, step=2)
        def _edges(i):
            @pl.when(i < m)
            def _slot0():
                pltpu.make_async_copy(x_hbm.at[src_idx.at[i]], rows0, sem0).wait()
                pltpu.async_copy(rows0, acc.at[dst_idx.at[i]], sem_sc0, add=True)

            @pl.when(i + 1 < m)
            def _slot1():
                pltpu.make_async_copy(x_hbm.at[src_idx.at[i + 1]], rows1,
                                      sem1).wait()
                pltpu.async_copy(rows1, acc.at[dst_idx.at[i + 1]], sem_sc1,
                                 add=True)

            @pl.when(i < m)
            def _drain0():
                pltpu.make_async_copy(rows0, acc.at[dst_idx.at[i]],
                                      sem_sc0).wait()

                @pl.when(i + 2 < m)
                def _():
                    pltpu.async_copy(x_hbm.at[src_idx.at[i + 2]], rows0, sem0)

            @pl.when(i + 1 < m)
            def _drain1():
                pltpu.make_async_copy(rows1, acc.at[dst_idx.at[i + 1]],
                                      sem_sc1).wait()

                @pl.when(i + 3 < m)
                def _():
                    pltpu.async_copy(x_hbm.at[src_idx.at[i + 3]], rows1, sem1)

    plsc.subcore_barrier()

    pltpu.sync_copy(acc.at[pl.ds(s * ROWS_PER_SUB, ROWS_PER_SUB)],
                    part_hbm.at[c].at[pl.ds(s * ROWS_PER_SUB, ROWS_PER_SUB)])


_sc_aggregate = pl.kernel(
    _sc_body,
    out_type=jax.ShapeDtypeStruct((NC, N_ACC, D), jnp.float32),
    mesh=_mesh,
    scratch_types=[
        pltpu.VMEM_SHARED((N_ACC, D), jnp.float32),
        pltpu.VMEM((HALF, CHUNK), jnp.int32),
        pltpu.VMEM((HALF, CHUNK), jnp.int32),
        pltpu.VMEM((CHUNK, D), jnp.float32),
        pltpu.VMEM((CHUNK, D), jnp.float32),
        pltpu.SemaphoreType.DMA,
        pltpu.SemaphoreType.DMA,
        pltpu.SemaphoreType.DMA,
        pltpu.SemaphoreType.DMA,
        pltpu.SemaphoreType.DMA,
        pltpu.SemaphoreType.DMA,
    ],
)


def _combine_body(p0, p1, w, o):
    o[...] = (p0[0] + p1[0]) * w[...]


def _combine(part, w2d):
    blk = 1000
    return pl.pallas_call(
        _combine_body,
        out_shape=jax.ShapeDtypeStruct((N_NODES, D), jnp.float32),
        grid=(N_NODES // blk,),
        in_specs=[
            pl.BlockSpec((1, blk, D), lambda i: (0, i, 0)),
            pl.BlockSpec((1, blk, D), lambda i: (1, i, 0)),
            pl.BlockSpec((1, D), lambda i: (0, 0)),
        ],
        out_specs=pl.BlockSpec((blk, D), lambda i: (i, 0)),
    )(part, part, w2d)


def kernel(feature_matrix, edge_index, W):
    n_pad = N_CHUNKS * CHUNK - N_EDGES
    src = jnp.concatenate(
        [edge_index[0], jnp.zeros((n_pad,), jnp.int32)]).reshape(N_CHUNKS, CHUNK)
    dst = jnp.concatenate(
        [edge_index[1], jnp.zeros((n_pad,), jnp.int32)]).reshape(N_CHUNKS, CHUNK)
    part = _sc_aggregate(feature_matrix, src, dst)
    return _combine(part, W.reshape(1, D))


# speedup vs baseline: 3.6194x; 1.1928x over previous
import jax
import jax.numpy as jnp
from jax import lax
from jax.experimental import pallas as pl
from jax.experimental.pallas import tpu as pltpu
from jax.experimental.pallas import tpu_sc as plsc

N_NODES = 10000
N_EDGES = 320000
D = 128

CHUNK = 128
HCHUNK = CHUNK // 2
SLOTS = ((0, 0), (0, 1), (1, 0), (1, 1))
NC = 2
NS = 16
N_TILES = NC * NS
CPT = 80
HALF = CPT // 2
N_CHUNKS = CPT * N_TILES
N_REAL = N_EDGES // CHUNK
N_ACC = 10112
ROWS_PER_SUB = N_ACC // NS

_mesh = plsc.VectorSubcoreMesh(core_axis_name="core", subcore_axis_name="subcore")


def _sc_body(x_hbm, src_hbm, dst_hbm, part_hbm,
             acc, src_idx, dst_idx, r0, r1, r2, r3,
             sem_s, sem_d, g0, g1, g2, g3, s0, s1, s2, s3):
    rows = (r0, r1, r2, r3)
    semg = (g0, g1, g2, g3)
    sems = (s0, s1, s2, s3)
    c = lax.axis_index("core")
    s = lax.axis_index("subcore")
    t = c * NS + s

    cp_s = pltpu.async_copy(src_hbm.at[pl.ds(t * CPT, HALF)], src_idx, sem_s)
    cp_d = pltpu.async_copy(dst_hbm.at[pl.ds(t * CPT, HALF)], dst_idx, sem_d)

    zero16 = jnp.zeros((16,), jnp.float32)

    @pl.loop(0, HCHUNK)
    def _zero_rows(i):
        for j in range(0, D, 16):
            rows[0][i, pl.ds(j, 16)] = zero16

    for k in range(ROWS_PER_SUB // HCHUNK):
        pltpu.sync_copy(rows[0], acc.at[pl.ds(s * ROWS_PER_SUB + k * HCHUNK,
                                              HCHUNK)])
    _tail = ROWS_PER_SUB % HCHUNK
    if _tail:
        pltpu.sync_copy(
            rows[0].at[pl.ds(0, _tail)],
            acc.at[pl.ds(s * ROWS_PER_SUB + ROWS_PER_SUB - _tail, _tail)])

    plsc.subcore_barrier()
    cp_s.wait()
    cp_d.wait()

    for h in range(CPT // HALF):
        m = jnp.clip(N_REAL - (t * CPT + h * HALF), 0, HALF)

        if h > 0:
            @pl.when(m > 0)
            def _reload():
                pltpu.sync_copy(src_hbm.at[pl.ds(t * CPT + h * HALF, HALF)],
                                src_idx)
                pltpu.sync_copy(dst_hbm.at[pl.ds(t * CPT + h * HALF, HALF)],
                                dst_idx)

        def _gidx(i, di, hf):
            return src_idx.at[i + di, pl.ds(hf * HCHUNK, HCHUNK)]

        def _sidx(i, di, hf):
            return dst_idx.at[i + di, pl.ds(hf * HCHUNK, HCHUNK)]

        for b, (di, hf) in enumerate(SLOTS):
            @pl.when(di < m)
            def _prime(b=b, di=di, hf=hf):
                pltpu.async_copy(x_hbm.at[_gidx(0, di, hf)], rows[b], semg[b])

        @pl.loop(0, HALF, step=2)
        def _edges(i):
            for b, (di, hf) in enumerate(SLOTS):
                @pl.when(i + di < m)
                def _slot(b=b, di=di, hf=hf):
                    pltpu.make_async_copy(x_hbm.at[_gidx(i, di, hf)], rows[b],
                                          semg[b]).wait()
                    pltpu.async_copy(rows[b], acc.at[_sidx(i, di, hf)],
                                     sems[b], add=True)

            for b, (di, hf) in enumerate(SLOTS):
                @pl.when(i + di < m)
                def _drain(b=b, di=di, hf=hf):
                    pltpu.make_async_copy(rows[b], acc.at[_sidx(i, di, hf)],
                                          sems[b]).wait()

                    @pl.when(i + di + 2 < m)
                    def _():
                        pltpu.async_copy(x_hbm.at[_gidx(i + 2, di, hf)],
                                         rows[b], semg[b])

    plsc.subcore_barrier()

    pltpu.sync_copy(acc.at[pl.ds(s * ROWS_PER_SUB, ROWS_PER_SUB)],
                    part_hbm.at[c].at[pl.ds(s * ROWS_PER_SUB, ROWS_PER_SUB)])


_sc_aggregate = pl.kernel(
    _sc_body,
    out_type=jax.ShapeDtypeStruct((NC, N_ACC, D), jnp.float32),
    mesh=_mesh,
    scratch_types=[
        pltpu.VMEM_SHARED((N_ACC, D), jnp.float32),
        pltpu.VMEM((HALF, CHUNK), jnp.int32),
        pltpu.VMEM((HALF, CHUNK), jnp.int32),
        pltpu.VMEM((HCHUNK, D), jnp.float32),
        pltpu.VMEM((HCHUNK, D), jnp.float32),
        pltpu.VMEM((HCHUNK, D), jnp.float32),
        pltpu.VMEM((HCHUNK, D), jnp.float32),
        pltpu.SemaphoreType.DMA,
        pltpu.SemaphoreType.DMA,
        pltpu.SemaphoreType.DMA,
        pltpu.SemaphoreType.DMA,
        pltpu.SemaphoreType.DMA,
        pltpu.SemaphoreType.DMA,
        pltpu.SemaphoreType.DMA,
        pltpu.SemaphoreType.DMA,
        pltpu.SemaphoreType.DMA,
        pltpu.SemaphoreType.DMA,
    ],
)


def _combine_body(p0, p1, w, o):
    o[...] = (p0[0] + p1[0]) * w[...]


def _combine(part, w2d):
    blk = 1000
    return pl.pallas_call(
        _combine_body,
        out_shape=jax.ShapeDtypeStruct((N_NODES, D), jnp.float32),
        grid=(N_NODES // blk,),
        in_specs=[
            pl.BlockSpec((1, blk, D), lambda i: (0, i, 0)),
            pl.BlockSpec((1, blk, D), lambda i: (1, i, 0)),
            pl.BlockSpec((1, D), lambda i: (0, 0)),
        ],
        out_specs=pl.BlockSpec((blk, D), lambda i: (i, 0)),
    )(part, part, w2d)


def kernel(feature_matrix, edge_index, W):
    n_pad = N_CHUNKS * CHUNK - N_EDGES
    src = jnp.concatenate(
        [edge_index[0], jnp.zeros((n_pad,), jnp.int32)]).reshape(N_CHUNKS, CHUNK)
    dst = jnp.concatenate(
        [edge_index[1], jnp.zeros((n_pad,), jnp.int32)]).reshape(N_CHUNKS, CHUNK)
    part = _sc_aggregate(feature_matrix, src, dst)
    return _combine(part, W.reshape(1, D))


# speedup vs baseline: 3.6891x; 1.0193x over previous
import jax
import jax.numpy as jnp
from jax import lax
from jax.experimental import pallas as pl
from jax.experimental.pallas import tpu as pltpu
from jax.experimental.pallas import tpu_sc as plsc

N_NODES = 10000
N_EDGES = 320000
D = 128

CHUNK = 128
NQ = 4
HCHUNK = CHUNK // NQ
NBUF = 2 * NQ
SLOTS = tuple((di, q) for di in (0, 1) for q in range(NQ))
NC = 2
NS = 16
N_TILES = NC * NS
CPT = 80
HALF = CPT // 2
N_CHUNKS = CPT * N_TILES
N_REAL = N_EDGES // CHUNK
N_ACC = 10112
ROWS_PER_SUB = N_ACC // NS

_mesh = plsc.VectorSubcoreMesh(core_axis_name="core", subcore_axis_name="subcore")


def _sc_body(x_hbm, src_hbm, dst_hbm, part_hbm,
             acc, src_idx, dst_idx, *rest):
    rows = rest[:NBUF]
    sem_s, sem_d = rest[NBUF], rest[NBUF + 1]
    semg = rest[NBUF + 2:2 * NBUF + 2]
    sems = rest[2 * NBUF + 2:3 * NBUF + 2]
    c = lax.axis_index("core")
    s = lax.axis_index("subcore")
    t = c * NS + s

    cp_s = pltpu.async_copy(src_hbm.at[pl.ds(t * CPT, HALF)], src_idx, sem_s)
    cp_d = pltpu.async_copy(dst_hbm.at[pl.ds(t * CPT, HALF)], dst_idx, sem_d)

    zero16 = jnp.zeros((16,), jnp.float32)

    @pl.loop(0, HCHUNK)
    def _zero_rows(i):
        for j in range(0, D, 16):
            rows[0][i, pl.ds(j, 16)] = zero16

    for k in range(ROWS_PER_SUB // HCHUNK):
        pltpu.sync_copy(rows[0], acc.at[pl.ds(s * ROWS_PER_SUB + k * HCHUNK,
                                              HCHUNK)])
    _tail = ROWS_PER_SUB % HCHUNK
    if _tail:
        pltpu.sync_copy(
            rows[0].at[pl.ds(0, _tail)],
            acc.at[pl.ds(s * ROWS_PER_SUB + ROWS_PER_SUB - _tail, _tail)])

    plsc.subcore_barrier()
    cp_s.wait()
    cp_d.wait()

    for h in range(CPT // HALF):
        m = jnp.clip(N_REAL - (t * CPT + h * HALF), 0, HALF)

        if h > 0:
            @pl.when(m > 0)
            def _reload():
                pltpu.sync_copy(src_hbm.at[pl.ds(t * CPT + h * HALF, HALF)],
                                src_idx)
                pltpu.sync_copy(dst_hbm.at[pl.ds(t * CPT + h * HALF, HALF)],
                                dst_idx)

        def _gidx(i, di, hf):
            return src_idx.at[i + di, pl.ds(hf * HCHUNK, HCHUNK)]

        def _sidx(i, di, hf):
            return dst_idx.at[i + di, pl.ds(hf * HCHUNK, HCHUNK)]

        for b, (di, hf) in enumerate(SLOTS):
            @pl.when(di < m)
            def _prime(b=b, di=di, hf=hf):
                pltpu.async_copy(x_hbm.at[_gidx(0, di, hf)], rows[b], semg[b])

        @pl.loop(0, HALF, step=2)
        def _edges(i):
            for b, (di, hf) in enumerate(SLOTS):
                @pl.when(i + di < m)
                def _slot(b=b, di=di, hf=hf):
                    pltpu.make_async_copy(x_hbm.at[_gidx(i, di, hf)], rows[b],
                                          semg[b]).wait()
                    pltpu.async_copy(rows[b], acc.at[_sidx(i, di, hf)],
                                     sems[b], add=True)

            for b, (di, hf) in enumerate(SLOTS):
                @pl.when(i + di < m)
                def _drain(b=b, di=di, hf=hf):
                    pltpu.make_async_copy(rows[b], acc.at[_sidx(i, di, hf)],
                                          sems[b]).wait()

                    @pl.when(i + di + 2 < m)
                    def _():
                        pltpu.async_copy(x_hbm.at[_gidx(i + 2, di, hf)],
                                         rows[b], semg[b])

    plsc.subcore_barrier()

    pltpu.sync_copy(acc.at[pl.ds(s * ROWS_PER_SUB, ROWS_PER_SUB)],
                    part_hbm.at[c].at[pl.ds(s * ROWS_PER_SUB, ROWS_PER_SUB)])


_sc_aggregate = pl.kernel(
    _sc_body,
    out_type=jax.ShapeDtypeStruct((NC, N_ACC, D), jnp.float32),
    mesh=_mesh,
    scratch_types=[
        pltpu.VMEM_SHARED((N_ACC, D), jnp.float32),
        pltpu.VMEM((HALF, CHUNK), jnp.int32),
        pltpu.VMEM((HALF, CHUNK), jnp.int32),
        *[pltpu.VMEM((HCHUNK, D), jnp.float32) for _ in range(NBUF)],
        pltpu.SemaphoreType.DMA,
        pltpu.SemaphoreType.DMA,
        *[pltpu.SemaphoreType.DMA for _ in range(NBUF)],
        *[pltpu.SemaphoreType.DMA for _ in range(NBUF)],
    ],
)


def _combine_body(p0, p1, w, o):
    o[...] = (p0[0] + p1[0]) * w[...]


def _combine(part, w2d):
    blk = 1000
    return pl.pallas_call(
        _combine_body,
        out_shape=jax.ShapeDtypeStruct((N_NODES, D), jnp.float32),
        grid=(N_NODES // blk,),
        in_specs=[
            pl.BlockSpec((1, blk, D), lambda i: (0, i, 0)),
            pl.BlockSpec((1, blk, D), lambda i: (1, i, 0)),
            pl.BlockSpec((1, D), lambda i: (0, 0)),
        ],
        out_specs=pl.BlockSpec((blk, D), lambda i: (i, 0)),
    )(part, part, w2d)


def kernel(feature_matrix, edge_index, W):
    n_pad = N_CHUNKS * CHUNK - N_EDGES
    src = jnp.concatenate(
        [edge_index[0], jnp.zeros((n_pad,), jnp.int32)]).reshape(N_CHUNKS, CHUNK)
    dst = jnp.concatenate(
        [edge_index[1], jnp.zeros((n_pad,), jnp.int32)]).reshape(N_CHUNKS, CHUNK)
    part = _sc_aggregate(feature_matrix, src, dst)
    return _combine(part, W.reshape(1, D))


# speedup vs baseline: 3.6990x; 1.0027x over previous
import jax
import jax.numpy as jnp
from jax import lax
from jax.experimental import pallas as pl
from jax.experimental.pallas import tpu as pltpu
from jax.experimental.pallas import tpu_sc as plsc

N_NODES = 10000
N_EDGES = 320000
D = 128

CHUNK = 128
NQ = 4
HCHUNK = CHUNK // NQ
NBUF = 2 * NQ
SLOTS = tuple((di, q) for di in (0, 1) for q in range(NQ))
NC = 2
NS = 16
N_TILES = NC * NS
CPT = 80
HALF = CPT // 2
N_CHUNKS = CPT * N_TILES
N_REAL = N_EDGES // CHUNK
N_ACC = 10112
ROWS_PER_SUB = N_ACC // NS

_mesh = plsc.VectorSubcoreMesh(core_axis_name="core", subcore_axis_name="subcore")


def _sc_body(x_hbm, src_hbm, dst_hbm, part_hbm,
             acc, src_idx, dst_idx, *rest):
    rows = rest[:NBUF]
    sem_s, sem_d = rest[NBUF], rest[NBUF + 1]
    semg = rest[NBUF + 2:2 * NBUF + 2]
    sems = rest[2 * NBUF + 2:3 * NBUF + 2]
    c = lax.axis_index("core")
    s = lax.axis_index("subcore")
    t = c * NS + s

    cp_s = pltpu.async_copy(src_hbm.at[pl.ds(t * CPT, HALF)], src_idx, sem_s)
    cp_d = pltpu.async_copy(dst_hbm.at[pl.ds(t * CPT, HALF)], dst_idx, sem_d)

    zero16 = jnp.zeros((16,), jnp.float32)

    @pl.loop(0, HCHUNK)
    def _zero_rows(i):
        for b in range(4):
            for j in range(0, D, 16):
                rows[b][i, pl.ds(j, 16)] = zero16

    _base = s * ROWS_PER_SUB
    _nfull = ROWS_PER_SUB // HCHUNK
    _tail = ROWS_PER_SUB % HCHUNK
    _zcopies = []
    for k in range(_nfull):
        _zcopies.append((rows[k % 4],
                         acc.at[pl.ds(_base + k * HCHUNK, HCHUNK)],
                         semg[k % NBUF]))
    if _tail:
        _zcopies.append((rows[0].at[pl.ds(0, _tail)],
                         acc.at[pl.ds(_base + _nfull * HCHUNK, _tail)],
                         sems[0]))
    for _src, _dst, _sem in _zcopies:
        pltpu.async_copy(_src, _dst, _sem)
    for _src, _dst, _sem in _zcopies:
        pltpu.make_async_copy(_src, _dst, _sem).wait()

    plsc.subcore_barrier()
    cp_s.wait()
    cp_d.wait()

    for h in range(CPT // HALF):
        m = jnp.clip(N_REAL - (t * CPT + h * HALF), 0, HALF)

        if h > 0:
            @pl.when(m > 0)
            def _reload():
                pltpu.sync_copy(src_hbm.at[pl.ds(t * CPT + h * HALF, HALF)],
                                src_idx)
                pltpu.sync_copy(dst_hbm.at[pl.ds(t * CPT + h * HALF, HALF)],
                                dst_idx)

        def _gidx(i, di, hf):
            return src_idx.at[i + di, pl.ds(hf * HCHUNK, HCHUNK)]

        def _sidx(i, di, hf):
            return dst_idx.at[i + di, pl.ds(hf * HCHUNK, HCHUNK)]

        for b, (di, hf) in enumerate(SLOTS):
            @pl.when(di < m)
            def _prime(b=b, di=di, hf=hf):
                pltpu.async_copy(x_hbm.at[_gidx(0, di, hf)], rows[b], semg[b])

        @pl.loop(0, HALF, step=2)
        def _edges(i):
            for b, (di, hf) in enumerate(SLOTS):
                @pl.when(i + di < m)
                def _slot(b=b, di=di, hf=hf):
                    pltpu.make_async_copy(x_hbm.at[_gidx(i, di, hf)], rows[b],
                                          semg[b]).wait()
                    pltpu.async_copy(rows[b], acc.at[_sidx(i, di, hf)],
                                     sems[b], add=True)

            for b, (di, hf) in enumerate(SLOTS):
                @pl.when(i + di < m)
                def _drain(b=b, di=di, hf=hf):
                    pltpu.make_async_copy(rows[b], acc.at[_sidx(i, di, hf)],
                                          sems[b]).wait()

                    @pl.when(i + di + 2 < m)
                    def _():
                        pltpu.async_copy(x_hbm.at[_gidx(i + 2, di, hf)],
                                         rows[b], semg[b])

    plsc.subcore_barrier()

    pltpu.sync_copy(acc.at[pl.ds(s * ROWS_PER_SUB, ROWS_PER_SUB)],
                    part_hbm.at[c].at[pl.ds(s * ROWS_PER_SUB, ROWS_PER_SUB)])


_sc_aggregate = pl.kernel(
    _sc_body,
    out_type=jax.ShapeDtypeStruct((NC, N_ACC, D), jnp.float32),
    mesh=_mesh,
    scratch_types=[
        pltpu.VMEM_SHARED((N_ACC, D), jnp.float32),
        pltpu.VMEM((HALF, CHUNK), jnp.int32),
        pltpu.VMEM((HALF, CHUNK), jnp.int32),
        *[pltpu.VMEM((HCHUNK, D), jnp.float32) for _ in range(NBUF)],
        pltpu.SemaphoreType.DMA,
        pltpu.SemaphoreType.DMA,
        *[pltpu.SemaphoreType.DMA for _ in range(NBUF)],
        *[pltpu.SemaphoreType.DMA for _ in range(NBUF)],
    ],
)


def _combine_body(p0, p1, w, o):
    o[...] = (p0[0] + p1[0]) * w[...]


def _combine(part, w2d):
    blk = 1000
    return pl.pallas_call(
        _combine_body,
        out_shape=jax.ShapeDtypeStruct((N_NODES, D), jnp.float32),
        grid=(N_NODES // blk,),
        in_specs=[
            pl.BlockSpec((1, blk, D), lambda i: (0, i, 0)),
            pl.BlockSpec((1, blk, D), lambda i: (1, i, 0)),
            pl.BlockSpec((1, D), lambda i: (0, 0)),
        ],
        out_specs=pl.BlockSpec((blk, D), lambda i: (i, 0)),
    )(part, part, w2d)


def kernel(feature_matrix, edge_index, W):
    n_pad = N_CHUNKS * CHUNK - N_EDGES
    src = jnp.concatenate(
        [edge_index[0], jnp.zeros((n_pad,), jnp.int32)]).reshape(N_CHUNKS, CHUNK)
    dst = jnp.concatenate(
        [edge_index[1], jnp.zeros((n_pad,), jnp.int32)]).reshape(N_CHUNKS, CHUNK)
    part = _sc_aggregate(feature_matrix, src, dst)
    return _combine(part, W.reshape(1, D))


# speedup vs baseline: 4.1446x; 1.1204x over previous
import jax
import jax.numpy as jnp
from jax import lax
from jax.experimental import pallas as pl
from jax.experimental.pallas import tpu as pltpu
from jax.experimental.pallas import tpu_sc as plsc

N_NODES = 10000
N_EDGES = 320000
D = 128

CHUNK = 128
NQ = 4
HCHUNK = CHUNK // NQ
NBUF = 2 * NQ
SLOTS = tuple((di, q) for di in (0, 1) for q in range(NQ))
NC = 2
NS = 16
N_TILES = NC * NS
CPT = 80
HALF = CPT // 2
N_CHUNKS = CPT * N_TILES
N_REAL = N_EDGES // CHUNK
N_ACC = 10112
ROWS_PER_SUB = N_ACC // NS

_mesh = plsc.VectorSubcoreMesh(core_axis_name="core", subcore_axis_name="subcore")


def _sc_body(x_hbm, src_hbm, dst_hbm, part_hbm,
             acc, src_idx, dst_idx, *rest):
    rows = rest[:NBUF]
    sem_s, sem_d = rest[NBUF], rest[NBUF + 1]
    semg = rest[NBUF + 2:2 * NBUF + 2]
    sems = rest[2 * NBUF + 2:3 * NBUF + 2]
    c = lax.axis_index("core")
    s = lax.axis_index("subcore")
    t = c * NS + s

    cp_s = pltpu.async_copy(src_hbm.at[pl.ds(t * CPT, HALF)], src_idx, sem_s)
    cp_d = pltpu.async_copy(dst_hbm.at[pl.ds(t * CPT, HALF)], dst_idx, sem_d)

    zero16 = jnp.zeros((16,), jnp.float32)

    @pl.loop(0, HCHUNK)
    def _zero_rows(i):
        for b in range(4):
            for j in range(0, D, 16):
                rows[b][i, pl.ds(j, 16)] = zero16

    _base = s * ROWS_PER_SUB
    _nfull = ROWS_PER_SUB // HCHUNK
    _tail = ROWS_PER_SUB % HCHUNK
    _zcopies = []
    for k in range(_nfull):
        _zcopies.append((rows[k % 4],
                         acc.at[pl.ds(_base + k * HCHUNK, HCHUNK)],
                         semg[k % NBUF]))
    if _tail:
        _zcopies.append((rows[0].at[pl.ds(0, _tail)],
                         acc.at[pl.ds(_base + _nfull * HCHUNK, _tail)],
                         sems[0]))
    for _src, _dst, _sem in _zcopies:
        pltpu.async_copy(_src, _dst, _sem)
    for _src, _dst, _sem in _zcopies:
        pltpu.make_async_copy(_src, _dst, _sem).wait()

    plsc.subcore_barrier()
    cp_s.wait()
    cp_d.wait()

    for h in range(CPT // HALF):
        m = jnp.clip(N_REAL - (t * CPT + h * HALF), 0, HALF)

        if h > 0:
            @pl.when(m > 0)
            def _reload():
                pltpu.sync_copy(src_hbm.at[pl.ds(t * CPT + h * HALF, HALF)],
                                src_idx)
                pltpu.sync_copy(dst_hbm.at[pl.ds(t * CPT + h * HALF, HALF)],
                                dst_idx)

        def _gidx(i, di, hf):
            return src_idx.at[i + di, pl.ds(hf * HCHUNK, HCHUNK)]

        def _sidx(i, di, hf):
            return dst_idx.at[i + di, pl.ds(hf * HCHUNK, HCHUNK)]

        for b, (di, hf) in enumerate(SLOTS):
            @pl.when(di < m)
            def _prime(b=b, di=di, hf=hf):
                pltpu.async_copy(x_hbm.at[_gidx(0, di, hf)], rows[b], semg[b])

        @pl.loop(0, HALF, step=2)
        def _edges(i):
            for di in (0, 1):
                for q in range(NQ):
                    b = di * NQ + q

                    @pl.when(i + di < m)
                    def _slot(b=b, di=di, hf=q):
                        pltpu.make_async_copy(x_hbm.at[_gidx(i, di, hf)],
                                              rows[b], semg[b]).wait()
                        pltpu.async_copy(rows[b], acc.at[_sidx(i, di, hf)],
                                         sems[b], add=True)

                for q in range(NQ):
                    b = di * NQ + q

                    @pl.when(i + di < m)
                    def _drain(b=b, di=di, hf=q):
                        pltpu.make_async_copy(rows[b], acc.at[_sidx(i, di, hf)],
                                              sems[b]).wait()

                        @pl.when(i + di + 2 < m)
                        def _():
                            pltpu.async_copy(x_hbm.at[_gidx(i + 2, di, hf)],
                                             rows[b], semg[b])

    plsc.subcore_barrier()

    pltpu.sync_copy(acc.at[pl.ds(s * ROWS_PER_SUB, ROWS_PER_SUB)],
                    part_hbm.at[c].at[pl.ds(s * ROWS_PER_SUB, ROWS_PER_SUB)])


_sc_aggregate = pl.kernel(
    _sc_body,
    out_type=jax.ShapeDtypeStruct((NC, N_ACC, D), jnp.float32),
    mesh=_mesh,
    scratch_types=[
        pltpu.VMEM_SHARED((N_ACC, D), jnp.float32),
        pltpu.VMEM((HALF, CHUNK), jnp.int32),
        pltpu.VMEM((HALF, CHUNK), jnp.int32),
        *[pltpu.VMEM((HCHUNK, D), jnp.float32) for _ in range(NBUF)],
        pltpu.SemaphoreType.DMA,
        pltpu.SemaphoreType.DMA,
        *[pltpu.SemaphoreType.DMA for _ in range(NBUF)],
        *[pltpu.SemaphoreType.DMA for _ in range(NBUF)],
    ],
)


def _combine_body(p0, p1, w, o):
    o[...] = (p0[0] + p1[0]) * w[...]


def _combine(part, w2d):
    blk = 1000
    return pl.pallas_call(
        _combine_body,
        out_shape=jax.ShapeDtypeStruct((N_NODES, D), jnp.float32),
        grid=(N_NODES // blk,),
        in_specs=[
            pl.BlockSpec((1, blk, D), lambda i: (0, i, 0)),
            pl.BlockSpec((1, blk, D), lambda i: (1, i, 0)),
            pl.BlockSpec((1, D), lambda i: (0, 0)),
        ],
        out_specs=pl.BlockSpec((blk, D), lambda i: (i, 0)),
    )(part, part, w2d)


def kernel(feature_matrix, edge_index, W):
    n_pad = N_CHUNKS * CHUNK - N_EDGES
    src = jnp.concatenate(
        [edge_index[0], jnp.zeros((n_pad,), jnp.int32)]).reshape(N_CHUNKS, CHUNK)
    dst = jnp.concatenate(
        [edge_index[1], jnp.zeros((n_pad,), jnp.int32)]).reshape(N_CHUNKS, CHUNK)
    part = _sc_aggregate(feature_matrix, src, dst)
    return _combine(part, W.reshape(1, D))


# speedup vs baseline: 4.1696x; 1.0060x over previous
import jax
import jax.numpy as jnp
from jax import lax
from jax.experimental import pallas as pl
from jax.experimental.pallas import tpu as pltpu
from jax.experimental.pallas import tpu_sc as plsc

N_NODES = 10000
N_EDGES = 320000
D = 128

CHUNK = 128
NQ = 4
HCHUNK = CHUNK // NQ
NBUF = 2 * NQ
SLOTS = tuple((di, q) for di in (0, 1) for q in range(NQ))
NC = 2
NS = 16
N_TILES = NC * NS
CPT = 80
HALF = CPT // 2
N_CHUNKS = CPT * N_TILES
N_REAL = N_EDGES // CHUNK
N_ACC = 10112
ROWS_PER_SUB = N_ACC // NS

_mesh = plsc.VectorSubcoreMesh(core_axis_name="core", subcore_axis_name="subcore")


def _sc_body(x_hbm, src_hbm, dst_hbm, part_hbm,
             acc, src_idx, dst_idx, *rest):
    rows = rest[:NBUF]
    sem_s, sem_d = rest[NBUF], rest[NBUF + 1]
    semg = rest[NBUF + 2:2 * NBUF + 2]
    sems = rest[2 * NBUF + 2:3 * NBUF + 2]
    c = lax.axis_index("core")
    s = lax.axis_index("subcore")
    t = c * NS + s

    cp_s = pltpu.async_copy(src_hbm.at[pl.ds(t * CPT, HALF)], src_idx, sem_s)
    cp_d = pltpu.async_copy(dst_hbm.at[pl.ds(t * CPT, HALF)], dst_idx, sem_d)

    zero16 = jnp.zeros((16,), jnp.float32)

    @pl.loop(0, HCHUNK)
    def _zero_rows(i):
        for b in range(4):
            for j in range(0, D, 16):
                rows[b][i, pl.ds(j, 16)] = zero16

    _base = s * ROWS_PER_SUB
    _nfull = ROWS_PER_SUB // HCHUNK
    _tail = ROWS_PER_SUB % HCHUNK
    _zcopies = []
    for k in range(_nfull):
        _zcopies.append((rows[k % 4],
                         acc.at[pl.ds(_base + k * HCHUNK, HCHUNK)],
                         semg[k % NBUF]))
    if _tail:
        _zcopies.append((rows[0].at[pl.ds(0, _tail)],
                         acc.at[pl.ds(_base + _nfull * HCHUNK, _tail)],
                         sems[0]))
    for _src, _dst, _sem in _zcopies:
        pltpu.async_copy(_src, _dst, _sem)
    for _src, _dst, _sem in _zcopies:
        pltpu.make_async_copy(_src, _dst, _sem).wait()

    plsc.subcore_barrier()
    cp_s.wait()
    cp_d.wait()

    for h in range(CPT // HALF):
        m = jnp.clip(N_REAL - (t * CPT + h * HALF), 0, HALF)

        if h > 0:
            @pl.when(m > 0)
            def _reload():
                pltpu.async_copy(src_hbm.at[pl.ds(t * CPT + h * HALF, HALF)],
                                 src_idx, sem_s)
                pltpu.async_copy(dst_hbm.at[pl.ds(t * CPT + h * HALF, HALF)],
                                 dst_idx, sem_d)
                pltpu.make_async_copy(src_hbm.at[pl.ds(t * CPT + h * HALF, HALF)],
                                      src_idx, sem_s).wait()
                pltpu.make_async_copy(dst_hbm.at[pl.ds(t * CPT + h * HALF, HALF)],
                                      dst_idx, sem_d).wait()

        def _gidx(i, di, hf):
            return src_idx.at[i + di, pl.ds(hf * HCHUNK, HCHUNK)]

        def _sidx(i, di, hf):
            return dst_idx.at[i + di, pl.ds(hf * HCHUNK, HCHUNK)]

        for b, (di, hf) in enumerate(SLOTS):
            @pl.when(di < m)
            def _prime(b=b, di=di, hf=hf):
                pltpu.async_copy(x_hbm.at[_gidx(0, di, hf)], rows[b], semg[b])

        @pl.loop(0, HALF, step=2)
        def _edges(i):
            for di in (0, 1):
                for q in range(NQ):
                    b = di * NQ + q

                    @pl.when(i + di < m)
                    def _slot(b=b, di=di, hf=q):
                        pltpu.make_async_copy(x_hbm.at[_gidx(i, di, hf)],
                                              rows[b], semg[b]).wait()
                        pltpu.async_copy(rows[b], acc.at[_sidx(i, di, hf)],
                                         sems[b], add=True)

                for q in range(NQ):
                    b = di * NQ + q

                    @pl.when(i + di < m)
                    def _drain(b=b, di=di, hf=q):
                        pltpu.make_async_copy(rows[b], acc.at[_sidx(i, di, hf)],
                                              sems[b]).wait()

                        @pl.when(i + di + 2 < m)
                        def _():
                            pltpu.async_copy(x_hbm.at[_gidx(i + 2, di, hf)],
                                             rows[b], semg[b])

    plsc.subcore_barrier()

    pltpu.sync_copy(acc.at[pl.ds(s * ROWS_PER_SUB, ROWS_PER_SUB)],
                    part_hbm.at[c].at[pl.ds(s * ROWS_PER_SUB, ROWS_PER_SUB)])


_sc_aggregate = pl.kernel(
    _sc_body,
    out_type=jax.ShapeDtypeStruct((NC, N_ACC, D), jnp.float32),
    mesh=_mesh,
    scratch_types=[
        pltpu.VMEM_SHARED((N_ACC, D), jnp.float32),
        pltpu.VMEM((HALF, CHUNK), jnp.int32),
        pltpu.VMEM((HALF, CHUNK), jnp.int32),
        *[pltpu.VMEM((HCHUNK, D), jnp.float32) for _ in range(NBUF)],
        pltpu.SemaphoreType.DMA,
        pltpu.SemaphoreType.DMA,
        *[pltpu.SemaphoreType.DMA for _ in range(NBUF)],
        *[pltpu.SemaphoreType.DMA for _ in range(NBUF)],
    ],
)


def _combine_body(p0, p1, w, o):
    o[...] = (p0[0] + p1[0]) * w[...]


def _combine(part, w2d):
    blk = 1000
    return pl.pallas_call(
        _combine_body,
        out_shape=jax.ShapeDtypeStruct((N_NODES, D), jnp.float32),
        grid=(N_NODES // blk,),
        in_specs=[
            pl.BlockSpec((1, blk, D), lambda i: (0, i, 0)),
            pl.BlockSpec((1, blk, D), lambda i: (1, i, 0)),
            pl.BlockSpec((1, D), lambda i: (0, 0)),
        ],
        out_specs=pl.BlockSpec((blk, D), lambda i: (i, 0)),
    )(part, part, w2d)


def kernel(feature_matrix, edge_index, W):
    n_pad = N_CHUNKS * CHUNK - N_EDGES
    src = jnp.concatenate(
        [edge_index[0], jnp.zeros((n_pad,), jnp.int32)]).reshape(N_CHUNKS, CHUNK)
    dst = jnp.concatenate(
        [edge_index[1], jnp.zeros((n_pad,), jnp.int32)]).reshape(N_CHUNKS, CHUNK)
    part = _sc_aggregate(feature_matrix, src, dst)
    return _combine(part, W.reshape(1, D))
